# baseline (speedup 1.0000x reference)
import jax
import jax.numpy as jnp
from jax import lax
from jax.experimental import pallas as pl
from jax.experimental.pallas import tpu as pltpu
from jax.experimental.pallas import tpu_sc as plsc

F32 = jnp.float32
NC, NS = 2, 16
BATCH = 128
SLAB = 13


def _plan(n_rows):
    nw = NC * NS
    rows_per = n_rows // nw
    tail = n_rows - rows_per * nw
    slabs = []
    r = 0
    while r < rows_per:
        nr = min(SLAB, rows_per - r)
        slabs.append((r, nr))
        r += nr
    return rows_per, tail, slabs


def _acc_geometry(n_nodes):
    per_sub = -(-n_nodes // NS)
    per_sub += (-per_sub) % 8
    return per_sub, per_sub * NS


def _zero_and_barrier(zbuf, accs, s, per_sub, h):
    def zloop(i, carry):
        zbuf[i] = jnp.zeros((h,), F32)
        return carry
    lax.fori_loop(0, per_sub, zloop, 0)
    sl = pl.ds(s * per_sub, per_sub)
    for acc in accs:
        pltpu.sync_copy(zbuf, acc.at[sl])
    plsc.subcore_barrier()
    return sl


def _sc_gather_segsum(n_nodes, n_rows, h):
    rows_per, tail, slabs = _plan(n_rows)
    per_sub, n_pad = _acc_geometry(n_nodes)
    mesh = plsc.VectorSubcoreMesh(core_axis_name="c", subcore_axis_name="s")
    scratch = [
        pltpu.VMEM((per_sub, h), F32),
        pltpu.VMEM((SLAB, 1, BATCH), jnp.int32),
        pltpu.VMEM((SLAB, 1, BATCH), jnp.int32),
        pltpu.VMEM((SLAB, 1, BATCH), jnp.int32),
        pltpu.VMEM((SLAB * BATCH, h), F32),
        pltpu.VMEM((SLAB * BATCH, h), F32),
        pltpu.SemaphoreType.DMA,
        pltpu.SemaphoreType.DMA,
        pltpu.VMEM_SHARED((n_pad, h), F32),
    ]

    def body(table, ei, outg, zbuf, didxa, didxb, sidx,
             rowsa, rowsb, semg, sems, accg):
        c = lax.axis_index("c")
        s = lax.axis_index("s")
        wid = c * NS + s
        sl = _zero_and_barrier(zbuf, [accg], s, per_sub, h)
        base = wid * rows_per

        pend = {}
        for ti, (r0, nr) in enumerate(slabs):
            buf = rowsa if ti % 2 == 0 else rowsb
            dbuf = didxa if ti % 2 == 0 else didxb
            if ti >= 2:
                for d in pend.pop(ti - 2):
                    d.wait()
            pltpu.sync_copy(ei.at[pl.ds(base + r0, nr), pl.ds(1, 1)],
                            dbuf.at[pl.ds(0, nr)])
            pltpu.sync_copy(ei.at[pl.ds(base + r0, nr), pl.ds(0, 1)],
                            sidx.at[pl.ds(0, nr)])
            gd = [pltpu.async_copy(table.at[sidx.at[j, 0]],
                                   buf.at[pl.ds(j * BATCH, BATCH)], semg)
                  for j in range(nr)]
            for d in gd:
                d.wait()
            pend[ti] = [pltpu.async_copy(buf.at[pl.ds(j * BATCH, BATCH)],
                                         accg.at[dbuf.at[j, 0]], sems,
                                         add=True)
                        for j in range(nr)]
        for ds in pend.values():
            for d in ds:
                d.wait()
        if tail:
            @pl.when(wid < tail)
            def _():
                r = NC * NS * rows_per + wid
                pltpu.sync_copy(ei.at[pl.ds(r, 1), pl.ds(1, 1)],
                                didxa.at[pl.ds(0, 1)])
                pltpu.sync_copy(ei.at[pl.ds(r, 1), pl.ds(0, 1)],
                                sidx.at[pl.ds(0, 1)])
                pltpu.async_copy(table.at[sidx.at[0, 0]],
                                 rowsa.at[pl.ds(0, BATCH)], semg).wait()
                pltpu.sync_copy(rowsa.at[pl.ds(0, BATCH)],
                                accg.at[didxa.at[0, 0]], add=True)

        plsc.subcore_barrier()
        pltpu.sync_copy(accg.at[sl], outg.at[c, sl])

    return pl.kernel(
        body,
        out_type=jax.ShapeDtypeStruct((NC, n_pad, h), F32),
        mesh=mesh, scratch_types=scratch,
        compiler_params=pltpu.CompilerParams(use_tc_tiling_on_sc=False))


def _sc_ea_segsum(n_nodes, n_rows, h, row_lo=0):
    rows_per, tail, slabs = _plan(n_rows)
    per_sub, n_pad = _acc_geometry(n_nodes)
    mesh = plsc.VectorSubcoreMesh(core_axis_name="c", subcore_axis_name="s")
    scratch = [
        pltpu.VMEM((per_sub, h), F32),
        pltpu.VMEM((SLAB, 1, BATCH), jnp.int32),
        pltpu.VMEM((SLAB, 1, BATCH), jnp.int32),
        pltpu.VMEM((SLAB * BATCH, h), F32),
        pltpu.VMEM((SLAB * BATCH, h), F32),
        pltpu.SemaphoreType.DMA,
        pltpu.SemaphoreType.DMA,
        pltpu.VMEM_SHARED((n_pad, h), F32),
    ]

    def body(ea, ei, oute, zbuf, didxa, didxb, ebufa, ebufb,
             semg, sems, acce):
        c = lax.axis_index("c")
        s = lax.axis_index("s")
        wid = c * NS + s
        sl = _zero_and_barrier(zbuf, [acce], s, per_sub, h)
        base = row_lo + wid * rows_per

        pend = {}
        for ti, (r0, nr) in enumerate(slabs):
            buf = ebufa if ti % 2 == 0 else ebufb
            dbuf = didxa if ti % 2 == 0 else didxb
            if ti >= 2:
                for d in pend.pop(ti - 2):
                    d.wait()
            pltpu.sync_copy(ei.at[pl.ds(base + r0, nr), pl.ds(1, 1)],
                            dbuf.at[pl.ds(0, nr)])
            pltpu.async_copy(
                ea.at[pl.ds((base + r0 - row_lo) * BATCH, nr * BATCH)],
                buf.at[pl.ds(0, nr * BATCH)], semg).wait()
            pend[ti] = [pltpu.async_copy(buf.at[pl.ds(j * BATCH, BATCH)],
                                         acce.at[dbuf.at[j, 0]], sems,
                                         add=True)
                        for j in range(nr)]
        for ds in pend.values():
            for d in ds:
                d.wait()
        if tail:
            @pl.when(wid < tail)
            def _():
                r = row_lo + NC * NS * rows_per + wid
                pltpu.sync_copy(ei.at[pl.ds(r, 1), pl.ds(1, 1)],
                                didxa.at[pl.ds(0, 1)])
                pltpu.sync_copy(ea.at[pl.ds((r - row_lo) * BATCH, BATCH)],
                                ebufa.at[pl.ds(0, BATCH)])
                pltpu.sync_copy(ebufa.at[pl.ds(0, BATCH)],
                                acce.at[didxa.at[0, 0]], add=True)

        plsc.subcore_barrier()
        pltpu.sync_copy(acce.at[sl], oute.at[c, sl])

    return pl.kernel(
        body,
        out_type=jax.ShapeDtypeStruct((NC, n_pad, h), F32),
        mesh=mesh, scratch_types=scratch,
        compiler_params=pltpu.CompilerParams(use_tc_tiling_on_sc=False))


def kernel(x, edge_index, edge_attr, W1n, W1e, b1, Ws1, bs1,
           W2n, W2e, b2, Ws2, bs2, W3, b3):
    N, D = x.shape
    E = edge_index.shape[1]
    DE = edge_attr.shape[1]
    H = W1n.shape[1]
    R = E // BATCH
    NW = N * H // 128
    n_pad = _acc_geometry(N)[1]
    NPW = n_pad * H // 128
    ei = edge_index.reshape(2, R, BATCH).transpose(1, 0, 2)

    eye8 = jnp.eye(8, dtype=F32)
    blk = lambda w: jnp.kron(eye8, w)
    tile8 = lambda b: jnp.tile(b, 8).reshape(1, 128)

    ean1 = edge_attr[:E // 2]
    ean2 = edge_attr[E // 2:]

    xg = x.reshape(NW, N * D // NW)

    def pre_body(x_ref, w1_ref, ws_ref, p1_ref, s1_ref):
        xv = x_ref[...]
        p1_ref[...] = jnp.dot(xv, w1_ref[...], preferred_element_type=F32)
        s1_ref[...] = jnp.dot(xv, ws_ref[...], preferred_element_type=F32)

    p1w, s1w = pl.pallas_call(
        pre_body,
        out_shape=[jax.ShapeDtypeStruct((NW, 128), F32)] * 2)(
            xg, blk(W1n), blk(Ws1))

    g1p = _sc_gather_segsum(N, R, H)(p1w.reshape(N, H), ei)
    eap1 = _sc_ea_segsum(N, R // 2, DE, 0)(ean1, ei)
    eap2 = _sc_ea_segsum(N, R // 2, DE, R // 2)(ean2, ei)

    def mid_body(g1_ref, ea_ref, eb_ref, s1_ref, w1e_ref, w2e_ref, w2n_ref,
                 ws2_ref, b1_ref, bs1_ref, b2_ref, bs2_ref, p2_ref, t_ref):
        ea = (ea_ref[0, :NW] + ea_ref[1, :NW]
              + eb_ref[0, :NW] + eb_ref[1, :NW])
        agg1 = (g1_ref[0, :NW] + g1_ref[1, :NW]
                + jnp.dot(ea, w1e_ref[...], preferred_element_type=F32)
                + b1_ref[...])
        hh = jnp.maximum(agg1 + s1_ref[...] + bs1_ref[...], 0.0)
        p2_ref[...] = jnp.dot(hh, w2n_ref[...], preferred_element_type=F32)
        t_ref[...] = (jnp.dot(ea, w2e_ref[...], preferred_element_type=F32)
                      + b2_ref[...]
                      + jnp.dot(hh, ws2_ref[...], preferred_element_type=F32)
                      + bs2_ref[...])

    p2w, tw = pl.pallas_call(
        mid_body,
        out_shape=[jax.ShapeDtypeStruct((NW, 128), F32)] * 2,
    )(g1p.reshape(NC, NPW, 128), eap1.reshape(NC, NPW, 128),
      eap2.reshape(NC, NPW, 128), s1w,
      blk(W1e), blk(W2e), blk(W2n), blk(Ws2),
      tile8(b1), tile8(bs1), tile8(b2), tile8(bs2))

    g2p = _sc_gather_segsum(N, R, H)(p2w.reshape(N, H), ei)

    def out_body(g2_ref, t_ref, w3_ref, b3_ref, o_ref):
        h2 = g2_ref[0, :NW] + g2_ref[1, :NW] + t_ref[...]
        o_ref[...] = (jnp.dot(h2, w3_ref[...], preferred_element_type=F32)
                      + b3_ref[...])

    ow = pl.pallas_call(
        out_body, out_shape=jax.ShapeDtypeStruct((NW, 8 * D), F32))(
            g2p.reshape(NC, NPW, 128), tw, blk(W3),
            jnp.tile(b3, 8).reshape(1, 8 * D))
    return ow.reshape(N, D)


# speedup vs baseline: 1.3867x; 1.3867x over previous
import jax
import jax.numpy as jnp
from jax import lax
from jax.experimental import pallas as pl
from jax.experimental.pallas import tpu as pltpu
from jax.experimental.pallas import tpu_sc as plsc

F32 = jnp.float32
NC, NS = 2, 16
BATCH = 128
SLAB = 13


def _plan(n_rows):
    nw = NC * NS
    rows_per = n_rows // nw
    tail = n_rows - rows_per * nw
    slabs = []
    r = 0
    while r < rows_per:
        nr = min(SLAB, rows_per - r)
        slabs.append((r, nr))
        r += nr
    return rows_per, tail, slabs


def _acc_geometry(n_nodes):
    per_sub = -(-n_nodes // NS)
    per_sub += (-per_sub) % 8
    return per_sub, per_sub * NS


def _zero_and_barrier(zbuf, accs, s, per_sub, h):
    def zloop(i, carry):
        zbuf[i] = jnp.zeros((h,), F32)
        return carry
    lax.fori_loop(0, per_sub, zloop, 0)
    sl = pl.ds(s * per_sub, per_sub)
    for acc in accs:
        pltpu.sync_copy(zbuf, acc.at[sl])
    plsc.subcore_barrier()
    return sl


def _sc_gather_segsum(n_nodes, n_rows, h):
    rows_per, tail, slabs = _plan(n_rows)
    per_sub, n_pad = _acc_geometry(n_nodes)
    mesh = plsc.VectorSubcoreMesh(core_axis_name="c", subcore_axis_name="s")
    scratch = [
        pltpu.VMEM((per_sub, h), F32),
        pltpu.VMEM((SLAB, 1, BATCH), jnp.int32),
        pltpu.VMEM((SLAB, 1, BATCH), jnp.int32),
        pltpu.VMEM((SLAB, 1, BATCH), jnp.int32),
        pltpu.VMEM((SLAB, 1, BATCH), jnp.int32),
        pltpu.VMEM((SLAB * BATCH, h), F32),
        pltpu.VMEM((SLAB * BATCH, h), F32),
        pltpu.SemaphoreType.DMA,
        pltpu.SemaphoreType.DMA,
        pltpu.SemaphoreType.DMA,
        pltpu.VMEM_SHARED((n_pad, h), F32),
    ]

    def body(table, ei, outg, zbuf, didxa, didxb, sidxa, sidxb,
             rowsa, rowsb, semga, semgb, sems, accg):
        c = lax.axis_index("c")
        s = lax.axis_index("s")
        wid = c * NS + s
        sl = _zero_and_barrier(zbuf, [accg], s, per_sub, h)
        base = wid * rows_per

        pend, gpend = {}, {}

        def fire_scatters(ti):
            nr = slabs[ti][1]
            buf = rowsa if ti % 2 == 0 else rowsb
            dbuf = didxa if ti % 2 == 0 else didxb
            for d in gpend.pop(ti):
                d.wait()
            pend[ti] = [pltpu.async_copy(buf.at[pl.ds(j * BATCH, BATCH)],
                                         accg.at[dbuf.at[j, 0]], sems,
                                         add=True)
                        for j in range(nr)]

        for ti, (r0, nr) in enumerate(slabs):
            buf = rowsa if ti % 2 == 0 else rowsb
            dbuf = didxa if ti % 2 == 0 else didxb
            sbuf = sidxa if ti % 2 == 0 else sidxb
            sg = semga if ti % 2 == 0 else semgb
            if ti >= 2:
                for d in pend.pop(ti - 2):
                    d.wait()
            pltpu.sync_copy(ei.at[pl.ds(base + r0, nr), pl.ds(1, 1)],
                            dbuf.at[pl.ds(0, nr)])
            pltpu.sync_copy(ei.at[pl.ds(base + r0, nr), pl.ds(0, 1)],
                            sbuf.at[pl.ds(0, nr)])
            gpend[ti] = [pltpu.async_copy(table.at[sbuf.at[j, 0]],
                                          buf.at[pl.ds(j * BATCH, BATCH)], sg)
                         for j in range(nr)]
            if ti >= 1:
                fire_scatters(ti - 1)
        fire_scatters(len(slabs) - 1)
        for ds in pend.values():
            for d in ds:
                d.wait()
        if tail:
            @pl.when(wid < tail)
            def _():
                r = NC * NS * rows_per + wid
                pltpu.sync_copy(ei.at[pl.ds(r, 1), pl.ds(1, 1)],
                                didxa.at[pl.ds(0, 1)])
                pltpu.sync_copy(ei.at[pl.ds(r, 1), pl.ds(0, 1)],
                                sidxa.at[pl.ds(0, 1)])
                pltpu.async_copy(table.at[sidxa.at[0, 0]],
                                 rowsa.at[pl.ds(0, BATCH)], semga).wait()
                pltpu.sync_copy(rowsa.at[pl.ds(0, BATCH)],
                                accg.at[didxa.at[0, 0]], add=True)

        plsc.subcore_barrier()
        pltpu.sync_copy(accg.at[sl], outg.at[c, sl])

    return pl.kernel(
        body,
        out_type=jax.ShapeDtypeStruct((NC, n_pad, h), F32),
        mesh=mesh, scratch_types=scratch,
        compiler_params=pltpu.CompilerParams(use_tc_tiling_on_sc=False))


def _sc_ea_segsum(n_nodes, n_rows, h, row_lo=0):
    rows_per, tail, slabs = _plan(n_rows)
    per_sub, n_pad = _acc_geometry(n_nodes)
    mesh = plsc.VectorSubcoreMesh(core_axis_name="c", subcore_axis_name="s")
    scratch = [
        pltpu.VMEM((per_sub, h), F32),
        pltpu.VMEM((SLAB, 1, BATCH), jnp.int32),
        pltpu.VMEM((SLAB, 1, BATCH), jnp.int32),
        pltpu.VMEM((SLAB * BATCH, h), F32),
        pltpu.VMEM((SLAB * BATCH, h), F32),
        pltpu.SemaphoreType.DMA,
        pltpu.SemaphoreType.DMA,
        pltpu.SemaphoreType.DMA,
        pltpu.VMEM_SHARED((n_pad, h), F32),
    ]

    def body(ea, ei, oute, zbuf, didxa, didxb, ebufa, ebufb,
             semga, semgb, sems, acce):
        c = lax.axis_index("c")
        s = lax.axis_index("s")
        wid = c * NS + s
        sl = _zero_and_barrier(zbuf, [acce], s, per_sub, h)
        base = row_lo + wid * rows_per

        pend, lpend = {}, {}

        def fire_scatters(ti):
            nr = slabs[ti][1]
            buf = ebufa if ti % 2 == 0 else ebufb
            dbuf = didxa if ti % 2 == 0 else didxb
            lpend.pop(ti).wait()
            pend[ti] = [pltpu.async_copy(buf.at[pl.ds(j * BATCH, BATCH)],
                                         acce.at[dbuf.at[j, 0]], sems,
                                         add=True)
                        for j in range(nr)]

        for ti, (r0, nr) in enumerate(slabs):
            buf = ebufa if ti % 2 == 0 else ebufb
            dbuf = didxa if ti % 2 == 0 else didxb
            sg = semga if ti % 2 == 0 else semgb
            if ti >= 2:
                for d in pend.pop(ti - 2):
                    d.wait()
            pltpu.sync_copy(ei.at[pl.ds(base + r0, nr), pl.ds(1, 1)],
                            dbuf.at[pl.ds(0, nr)])
            lpend[ti] = pltpu.async_copy(
                ea.at[pl.ds((base + r0 - row_lo) * BATCH, nr * BATCH)],
                buf.at[pl.ds(0, nr * BATCH)], sg)
            if ti >= 1:
                fire_scatters(ti - 1)
        fire_scatters(len(slabs) - 1)
        for ds in pend.values():
            for d in ds:
                d.wait()
        if tail:
            @pl.when(wid < tail)
            def _():
                r = row_lo + NC * NS * rows_per + wid
                pltpu.sync_copy(ei.at[pl.ds(r, 1), pl.ds(1, 1)],
                                didxa.at[pl.ds(0, 1)])
                pltpu.sync_copy(ea.at[pl.ds((r - row_lo) * BATCH, BATCH)],
                                ebufa.at[pl.ds(0, BATCH)])
                pltpu.sync_copy(ebufa.at[pl.ds(0, BATCH)],
                                acce.at[didxa.at[0, 0]], add=True)

        plsc.subcore_barrier()
        pltpu.sync_copy(acce.at[sl], oute.at[c, sl])

    return pl.kernel(
        body,
        out_type=jax.ShapeDtypeStruct((NC, n_pad, h), F32),
        mesh=mesh, scratch_types=scratch,
        compiler_params=pltpu.CompilerParams(use_tc_tiling_on_sc=False))


def kernel(x, edge_index, edge_attr, W1n, W1e, b1, Ws1, bs1,
           W2n, W2e, b2, Ws2, bs2, W3, b3):
    N, D = x.shape
    E = edge_index.shape[1]
    DE = edge_attr.shape[1]
    H = W1n.shape[1]
    R = E // BATCH
    NW = N * H // 128
    n_pad = _acc_geometry(N)[1]
    NPW = n_pad * H // 128
    ei = edge_index.reshape(2, R, BATCH).transpose(1, 0, 2)

    eye8 = jnp.eye(8, dtype=F32)
    blk = lambda w: jnp.kron(eye8, w)
    tile8 = lambda b: jnp.tile(b, 8).reshape(1, 128)

    ean = edge_attr

    xg = x.reshape(NW, N * D // NW)

    def pre_body(x_ref, w1_ref, ws_ref, p1_ref, s1_ref):
        xv = x_ref[...]
        p1_ref[...] = jnp.dot(xv, w1_ref[...], preferred_element_type=F32)
        s1_ref[...] = jnp.dot(xv, ws_ref[...], preferred_element_type=F32)

    p1w, s1w = pl.pallas_call(
        pre_body,
        out_shape=[jax.ShapeDtypeStruct((NW, 128), F32)] * 2)(
            xg, blk(W1n), blk(Ws1))

    g1p = _sc_gather_segsum(N, R, H)(p1w.reshape(N, H), ei)
    eap = _sc_ea_segsum(N, R, DE)(ean, ei)

    def mid_body(g1_ref, ea_ref, s1_ref, w1e_ref, w2e_ref, w2n_ref,
                 ws2_ref, b1_ref, bs1_ref, b2_ref, bs2_ref, p2_ref, t_ref):
        ea = ea_ref[0, :NW] + ea_ref[1, :NW]
        agg1 = (g1_ref[0, :NW] + g1_ref[1, :NW]
                + jnp.dot(ea, w1e_ref[...], preferred_element_type=F32)
                + b1_ref[...])
        hh = jnp.maximum(agg1 + s1_ref[...] + bs1_ref[...], 0.0)
        p2_ref[...] = jnp.dot(hh, w2n_ref[...], preferred_element_type=F32)
        t_ref[...] = (jnp.dot(ea, w2e_ref[...], preferred_element_type=F32)
                      + b2_ref[...]
                      + jnp.dot(hh, ws2_ref[...], preferred_element_type=F32)
                      + bs2_ref[...])

    p2w, tw = pl.pallas_call(
        mid_body,
        out_shape=[jax.ShapeDtypeStruct((NW, 128), F32)] * 2,
    )(g1p.reshape(NC, NPW, 128), eap.reshape(NC, NPW, 128), s1w,
      blk(W1e), blk(W2e), blk(W2n), blk(Ws2),
      tile8(b1), tile8(bs1), tile8(b2), tile8(bs2))

    g2p = _sc_gather_segsum(N, R, H)(p2w.reshape(N, H), ei)

    def out_body(g2_ref, t_ref, w3_ref, b3_ref, o_ref):
        h2 = g2_ref[0, :NW] + g2_ref[1, :NW] + t_ref[...]
        o_ref[...] = (jnp.dot(h2, w3_ref[...], preferred_element_type=F32)
                      + b3_ref[...])

    ow = pl.pallas_call(
        out_body, out_shape=jax.ShapeDtypeStruct((NW, 8 * D), F32))(
            g2p.reshape(NC, NPW, 128), tw, blk(W3),
            jnp.tile(b3, 8).reshape(1, 8 * D))
    return ow.reshape(N, D)
